# baseline (speedup 1.0000x reference)
import functools

import jax
import jax.numpy as jnp
from jax import lax
from jax.experimental import pallas as pl
from jax.experimental.pallas import tpu as pltpu
from jax.experimental.pallas import tpu_sc as plsc

F32 = jnp.float32


def _mm(a, b):
    return lax.dot_general(a, b, (((1,), (0,)), ((), ())),
                           preferred_element_type=F32)


def _mm_t(a, b):
    return lax.dot_general(a, b, (((1,), (1,)), ((), ())),
                           preferred_element_type=F32)


def _lrelu(x):
    return jnp.where(x >= 0, x, 0.2 * x)


def _tile(n, cap):
    for t in (512, 256, 128, 64, 32, 16, 8):
        if t <= cap and n % t == 0:
            return t
    raise ValueError(f"no tile for {n}")


def _node_linear(x, w1t, w2t, bias):
    n, d = x.shape
    m = w1t.shape[1]
    t = _tile(n, 512)

    def body(x_ref, w1_ref, w2_ref, b_ref, a_ref, bo_ref):
        xt = x_ref[...]
        a_ref[...] = _mm(xt, w1_ref[...])
        bo_ref[...] = _mm(xt, w2_ref[...]) + b_ref[...]

    return pl.pallas_call(
        body,
        grid=(n // t,),
        in_specs=[pl.BlockSpec((t, d), lambda i: (i, 0)),
                  pl.BlockSpec((d, m), lambda i: (0, 0)),
                  pl.BlockSpec((d, m), lambda i: (0, 0)),
                  pl.BlockSpec((1, m), lambda i: (0, 0))],
        out_specs=[pl.BlockSpec((t, m), lambda i: (i, 0)),
                   pl.BlockSpec((t, m), lambda i: (i, 0))],
        out_shape=[jax.ShapeDtypeStruct((n, m), F32),
                   jax.ShapeDtypeStruct((n, m), F32)],
    )(x, w1t, w2t, bias)


def _edge_combine(gath, bnode, ec1t, ec1b):
    kk, n, d2 = gath.shape
    o = ec1t.shape[1]
    t = _tile(n, 512)

    def body(g_ref, b_ref, w_ref, bias_ref, f1_ref, fmax_ref, fsum_ref):
        i = pl.program_id(0)
        bn = b_ref[...]
        w = w_ref[...]
        bias = bias_ref[...]
        acc = None
        for k in range(kk):
            f = _mm(_lrelu(g_ref[k] + bn), w) + bias
            acc = f if acc is None else jnp.maximum(acc, f)
        f1_ref[...] = acc
        tmax = jnp.max(acc, axis=0, keepdims=True)
        tsum = jnp.sum(acc, axis=0, keepdims=True)

        @pl.when(i == 0)
        def _():
            fmax_ref[...] = tmax
            fsum_ref[...] = tsum

        @pl.when(i != 0)
        def _():
            fmax_ref[...] = jnp.maximum(fmax_ref[...], tmax)
            fsum_ref[...] = fsum_ref[...] + tsum

    return pl.pallas_call(
        body,
        grid=(n // t,),
        in_specs=[pl.BlockSpec((kk, t, d2), lambda i: (0, i, 0)),
                  pl.BlockSpec((t, d2), lambda i: (i, 0)),
                  pl.BlockSpec((d2, o), lambda i: (0, 0)),
                  pl.BlockSpec((1, o), lambda i: (0, 0))],
        out_specs=[pl.BlockSpec((t, o), lambda i: (i, 0)),
                   pl.BlockSpec((1, o), lambda i: (0, 0)),
                   pl.BlockSpec((1, o), lambda i: (0, 0))],
        out_shape=[jax.ShapeDtypeStruct((n, o), F32),
                   jax.ShapeDtypeStruct((1, o), F32),
                   jax.ShapeDtypeStruct((1, o), F32)],
    )(gath, bnode, ec1t, ec1b)


def _knn_topk(x, kp1):
    n, d = x.shape
    r = _tile(n, 256)
    c = _tile(n, 512)
    nct = n // c
    neg = -3.0e38
    bigi = 2**30

    def body(xf_ref, xr_ref, out_ref):
        xr = xr_ref[...]
        xxr = jnp.sum(xr * xr, axis=1, keepdims=True)
        iota = lax.broadcasted_iota(jnp.int32, (r, c), 1)

        def col_step(tt, carry):
            tv, ti = carry
            xc = xf_ref[pl.ds(tt * c, c), :]
            xxc = jnp.sum(xc * xc, axis=1)
            v = 2.0 * _mm_t(xr, xc) - xxr - xxc[None, :]
            base = tt * c
            lv, li = [], []
            for _ in range(kp1):
                m = jnp.max(v, axis=1, keepdims=True)
                cand = jnp.where(v == m, iota, bigi)
                j = jnp.min(cand, axis=1, keepdims=True)
                lv.append(m)
                li.append(j + base)
                v = jnp.where(iota == j, neg, v)
            mv = jnp.concatenate([tv] + lv, axis=1)
            mi = jnp.concatenate([ti] + li, axis=1)
            nv, ni = [], []
            for _ in range(kp1):
                m = jnp.max(mv, axis=1, keepdims=True)
                cand = jnp.where(mv == m, mi, bigi)
                j = jnp.min(cand, axis=1, keepdims=True)
                nv.append(m)
                ni.append(j)
                mv = jnp.where((mv == m) & (mi == j), neg, mv)
            return (jnp.concatenate(nv, axis=1), jnp.concatenate(ni, axis=1))

        tv0 = jnp.full((r, kp1), neg, F32)
        ti0 = jnp.full((r, kp1), bigi, jnp.int32)
        _, ti = lax.fori_loop(0, nct, col_step, (tv0, ti0))
        out_ref[...] = ti

    return pl.pallas_call(
        body,
        grid=(n // r,),
        in_specs=[pl.BlockSpec((n, d), lambda i: (0, 0)),
                  pl.BlockSpec((r, d), lambda i: (i, 0))],
        out_specs=pl.BlockSpec((r, kp1), lambda i: (i, 0)),
        out_shape=jax.ShapeDtypeStruct((n, kp1), jnp.int32),
    )(x, x)


def _mlp0(feat, f1, fmax, fsum, w, e1, e2, eb):
    n, d0 = feat.shape
    o = f1.shape[1]
    m1 = e1.shape[1]
    t = _tile(n, 512)
    (w0a, w0b, w0c, w0d, b0, w1, b1, w2, b2, w3, b3) = w

    def body(feat_ref, f1_ref, fmax_ref, fsum_ref, w0a_r, w0b_r, w0c_r,
             w0d_r, b0_r, w1_r, b1_r, w2_r, b2_r, w3_r, b3_r, e1_r, e2_r,
             eb_r, o0_ref, a1_ref, b1o_ref):
        favg = fsum_ref[...] / n
        cst = _mm(fmax_ref[...], w0c_r[...]) + _mm(favg, w0d_r[...])
        y = _lrelu(_mm(feat_ref[...], w0a_r[...]) +
                   _mm(f1_ref[...], w0b_r[...]) + cst + b0_r[...])
        y = _lrelu(_mm(y, w1_r[...]) + b1_r[...])
        y = _lrelu(_mm(y, w2_r[...]) + b2_r[...])
        o0 = _mm(y, w3_r[...]) + b3_r[...]
        o0_ref[...] = o0
        a1_ref[...] = _mm(o0, e1_r[...])
        b1o_ref[...] = _mm(o0, e2_r[...]) + eb_r[...]

    full = lambda a: pl.BlockSpec(a.shape, lambda i: (0,) * a.ndim)
    return pl.pallas_call(
        body,
        grid=(n // t,),
        in_specs=[pl.BlockSpec((t, d0), lambda i: (i, 0)),
                  pl.BlockSpec((t, o), lambda i: (i, 0)),
                  full(fmax), full(fsum), full(w0a), full(w0b), full(w0c),
                  full(w0d), full(b0), full(w1), full(b1), full(w2),
                  full(b2), full(w3), full(b3), full(e1), full(e2),
                  full(eb)],
        out_specs=[pl.BlockSpec((t, o), lambda i: (i, 0)),
                   pl.BlockSpec((t, m1), lambda i: (i, 0)),
                   pl.BlockSpec((t, m1), lambda i: (i, 0))],
        out_shape=[jax.ShapeDtypeStruct((n, o), F32),
                   jax.ShapeDtypeStruct((n, m1), F32),
                   jax.ShapeDtypeStruct((n, m1), F32)],
    )(feat, f1, fmax, fsum, w0a, w0b, w0c, w0d, b0, w1, b1, w2, b2, w3,
      b3, e1, e2, eb)


def _mlp1(o0, f1, fmax, fsum, w):
    n, d0 = o0.shape
    o = f1.shape[1]
    t = _tile(n, 512)
    (w0a, w0b, w0c, w0d, b0, w1, b1, w2, b2, w3, b3) = w

    def body(o0_ref, f1_ref, fmax_ref, fsum_ref, w0a_r, w0b_r, w0c_r,
             w0d_r, b0_r, w1_r, b1_r, w2_r, b2_r, w3_r, b3_r, o1_ref):
        favg = fsum_ref[...] / n
        cst = _mm(fmax_ref[...], w0c_r[...]) + _mm(favg, w0d_r[...])
        y = _lrelu(_mm(o0_ref[...], w0a_r[...]) +
                   _mm(f1_ref[...], w0b_r[...]) + cst + b0_r[...])
        y = _lrelu(_mm(y, w1_r[...]) + b1_r[...])
        y = _lrelu(_mm(y, w2_r[...]) + b2_r[...])
        o1_ref[...] = _mm(y, w3_r[...]) + b3_r[...]

    full = lambda a: pl.BlockSpec(a.shape, lambda i: (0,) * a.ndim)
    return pl.pallas_call(
        body,
        grid=(n // t,),
        in_specs=[pl.BlockSpec((t, d0), lambda i: (i, 0)),
                  pl.BlockSpec((t, o), lambda i: (i, 0)),
                  full(fmax), full(fsum), full(w0a), full(w0b), full(w0c),
                  full(w0d), full(b0), full(w1), full(b1), full(w2),
                  full(b2), full(w3), full(b3)],
        out_specs=pl.BlockSpec((t, o), lambda i: (i, 0)),
        out_shape=jax.ShapeDtypeStruct((n, o), F32),
    )(o0, f1, fmax, fsum, w0a, w0b, w0c, w0d, b0, w1, b1, w2, b2, w3, b3)


def _gather_rows(table, idx):
    nrows, d = table.shape
    m = idx.shape[0]
    info = plsc.get_sparse_core_info()
    nw = info.num_cores * info.num_subcores
    nc = info.num_cores
    bpw = m // nw
    assert bpw * nw == m
    max_rows = max(8, (220 * 1024) // (d * 4))
    chunk = 0
    for cand in range(min(bpw, max_rows), 7, -1):
        if cand % 8 == 0 and bpw % cand == 0:
            chunk = cand
            break
    assert chunk, (bpw, max_rows)
    nck = bpw // chunk

    mesh = plsc.VectorSubcoreMesh(core_axis_name="c", subcore_axis_name="s")

    @functools.partial(
        pl.kernel, mesh=mesh,
        compiler_params=pltpu.CompilerParams(use_tc_tiling_on_sc=False),
        out_type=jax.ShapeDtypeStruct((m, d), F32),
        scratch_types=[pltpu.VMEM((nck, chunk), jnp.int32),
                       pltpu.VMEM((chunk, d), F32),
                       pltpu.SemaphoreType.DMA],
    )
    def gk(table_hbm, idx_hbm, out_hbm, idx_v, rows_v, sem):
        wid = lax.axis_index("s") * nc + lax.axis_index("c")
        pltpu.sync_copy(idx_hbm.at[wid], idx_v)
        base = wid * bpw
        for ck in range(nck):
            pltpu.async_copy(table_hbm.at[idx_v.at[ck]], rows_v, sem).wait()
            pltpu.sync_copy(rows_v,
                            out_hbm.at[pl.ds(base + ck * chunk, chunk)])

    return gk(table, idx.reshape(nw, nck, chunk))


def kernel(feat, idx0, params):
    b, n, d0 = feat.shape
    k = idx0.shape[2]
    eps = 1e-5

    def fold_edge(p, d):
        s = p['bne0_g'] / jnp.sqrt(1.0 + eps)
        w1 = p['ec0_w'][:, :d] * s[:, None]
        w2 = p['ec0_w'][:, d:] * s[:, None]
        return w1.T, (w2 - w1).T, p['bne0_b'][None, :]

    def fold_fc(wname, p, bn):
        if bn is None:
            return p[wname].T, p[wname.replace('_w', '_b')][None, :]
        s = p[bn + '_g'] / jnp.sqrt(1.0 + eps)
        return (p[wname] * s[:, None]).T, p[bn + '_b'][None, :]

    def mlp_weights(p, d, o):
        w0t, b0 = fold_fc('fc0_w', p, 'bn0')
        w1t, b1 = fold_fc('fc1_w', p, 'bn1')
        w2t, b2 = fold_fc('fc2_w', p, 'bn2')
        w3t, b3 = fold_fc('fc3_w', p, None)
        return (w0t[:d], w0t[d:d + o], w0t[d + o:d + 2 * o],
                w0t[d + 2 * o:], b0, w1t, b1, w2t, b2, w3t, b3)

    p0, p1 = params['b0'], params['b1']
    o0_dim, o1_dim = p0['fc3_w'].shape[0], p1['fc3_w'].shape[0]
    x0 = feat[0]

    w1t0, w2t0, eb0 = fold_edge(p0, d0)
    a0, b0n = _node_linear(x0, w1t0, w2t0, eb0)
    idxf0 = idx0[0].T.reshape(-1)
    g0 = _gather_rows(a0, idxf0).reshape(k, n, a0.shape[1])
    f1_0, fm0, fs0 = _edge_combine(g0, b0n, p0['ec1_w'].T,
                                   p0['ec1_b'][None, :])
    w1t1, w2t1, eb1 = fold_edge(p1, o0_dim)
    o0, a1, b1n = _mlp0(x0, f1_0, fm0, fs0, mlp_weights(p0, d0, o0_dim),
                        w1t1, w2t1, eb1)

    idx1 = _knn_topk(o0, k + 1)[:, 1:]
    idxf1 = idx1.T.reshape(-1)
    g1 = _gather_rows(a1, idxf1).reshape(k, n, a1.shape[1])
    f1_1, fm1, fs1 = _edge_combine(g1, b1n, p1['ec1_w'].T,
                                   p1['ec1_b'][None, :])
    o1 = _mlp1(o0, f1_1, fm1, fs1, mlp_weights(p1, o0_dim, o1_dim))

    return jnp.concatenate([o0, o1], axis=1)[None]


# speedup vs baseline: 2.6873x; 2.6873x over previous
import functools

import jax
import jax.numpy as jnp
from jax import lax
from jax.experimental import pallas as pl
from jax.experimental.pallas import tpu as pltpu
from jax.experimental.pallas import tpu_sc as plsc

F32 = jnp.float32


def _mm(a, b):
    return lax.dot_general(a, b, (((1,), (0,)), ((), ())),
                           preferred_element_type=F32)


def _mm_t(a, b):
    return lax.dot_general(a, b, (((1,), (1,)), ((), ())),
                           preferred_element_type=F32)


def _lrelu(x):
    return jnp.where(x >= 0, x, 0.2 * x)


def _tile(n, cap):
    for t in (512, 256, 128, 64, 32, 16, 8):
        if t <= cap and n % t == 0:
            return t
    raise ValueError(f"no tile for {n}")


def _node_linear(x, w1t, w2t, bias):
    n, d = x.shape
    m = w1t.shape[1]
    t = _tile(n, 512)

    def body(x_ref, w1_ref, w2_ref, b_ref, a_ref, bo_ref):
        xt = x_ref[...]
        a_ref[...] = _mm(xt, w1_ref[...])
        bo_ref[...] = _mm(xt, w2_ref[...]) + b_ref[...]

    return pl.pallas_call(
        body,
        grid=(n // t,),
        in_specs=[pl.BlockSpec((t, d), lambda i: (i, 0)),
                  pl.BlockSpec((d, m), lambda i: (0, 0)),
                  pl.BlockSpec((d, m), lambda i: (0, 0)),
                  pl.BlockSpec((1, m), lambda i: (0, 0))],
        out_specs=[pl.BlockSpec((t, m), lambda i: (i, 0)),
                   pl.BlockSpec((t, m), lambda i: (i, 0))],
        out_shape=[jax.ShapeDtypeStruct((n, m), F32),
                   jax.ShapeDtypeStruct((n, m), F32)],
    )(x, w1t, w2t, bias)


def _edge_combine(gath, bnode, ec1t, ec1b):
    kk, n, d2 = gath.shape
    o = ec1t.shape[1]
    t = _tile(n, 512)

    def body(g_ref, b_ref, w_ref, bias_ref, f1_ref, fmax_ref, fsum_ref):
        i = pl.program_id(0)
        bn = b_ref[...]
        w = w_ref[...]
        bias = bias_ref[...]
        acc = None
        for k in range(kk):
            f = _mm(_lrelu(g_ref[k] + bn), w) + bias
            acc = f if acc is None else jnp.maximum(acc, f)
        f1_ref[...] = acc
        tmax = jnp.max(acc, axis=0, keepdims=True)
        tsum = jnp.sum(acc, axis=0, keepdims=True)

        @pl.when(i == 0)
        def _():
            fmax_ref[...] = tmax
            fsum_ref[...] = tsum

        @pl.when(i != 0)
        def _():
            fmax_ref[...] = jnp.maximum(fmax_ref[...], tmax)
            fsum_ref[...] = fsum_ref[...] + tsum

    return pl.pallas_call(
        body,
        grid=(n // t,),
        in_specs=[pl.BlockSpec((kk, t, d2), lambda i: (0, i, 0)),
                  pl.BlockSpec((t, d2), lambda i: (i, 0)),
                  pl.BlockSpec((d2, o), lambda i: (0, 0)),
                  pl.BlockSpec((1, o), lambda i: (0, 0))],
        out_specs=[pl.BlockSpec((t, o), lambda i: (i, 0)),
                   pl.BlockSpec((1, o), lambda i: (0, 0)),
                   pl.BlockSpec((1, o), lambda i: (0, 0))],
        out_shape=[jax.ShapeDtypeStruct((n, o), F32),
                   jax.ShapeDtypeStruct((1, o), F32),
                   jax.ShapeDtypeStruct((1, o), F32)],
    )(gath, bnode, ec1t, ec1b)


def _knn_topk(x, kp1):
    n, d = x.shape
    r = _tile(n, 128)
    neg = -3.0e38
    bigi = 2**30

    def body(xf_ref, xr_ref, out_ref):
        xr = xr_ref[...]
        xxr = jnp.sum(xr * xr, axis=1, keepdims=True)
        xf = xf_ref[...]
        xxc = jnp.sum(xf * xf, axis=1)
        v = 2.0 * _mm_t(xr, xf) - xxr - xxc[None, :]
        iota = lax.broadcasted_iota(jnp.int32, (r, n), 1)
        ni = []
        for _ in range(kp1):
            m = jnp.max(v, axis=1, keepdims=True)
            cand = jnp.where(v == m, iota, bigi)
            j = jnp.min(cand, axis=1, keepdims=True)
            ni.append(j)
            v = jnp.where(iota == j, neg, v)
        out_ref[...] = jnp.concatenate(ni, axis=1)

    return pl.pallas_call(
        body,
        grid=(n // r,),
        in_specs=[pl.BlockSpec((n, d), lambda i: (0, 0)),
                  pl.BlockSpec((r, d), lambda i: (i, 0))],
        out_specs=pl.BlockSpec((r, kp1), lambda i: (i, 0)),
        out_shape=jax.ShapeDtypeStruct((n, kp1), jnp.int32),
    )(x, x)


def _mlp0(feat, f1, fmax, fsum, w, e1, e2, eb):
    n, d0 = feat.shape
    o = f1.shape[1]
    m1 = e1.shape[1]
    t = _tile(n, 512)
    (w0a, w0b, w0c, w0d, b0, w1, b1, w2, b2, w3, b3) = w

    def body(feat_ref, f1_ref, fmax_ref, fsum_ref, w0a_r, w0b_r, w0c_r,
             w0d_r, b0_r, w1_r, b1_r, w2_r, b2_r, w3_r, b3_r, e1_r, e2_r,
             eb_r, o0_ref, a1_ref, b1o_ref):
        favg = fsum_ref[...] / n
        cst = _mm(fmax_ref[...], w0c_r[...]) + _mm(favg, w0d_r[...])
        y = _lrelu(_mm(feat_ref[...], w0a_r[...]) +
                   _mm(f1_ref[...], w0b_r[...]) + cst + b0_r[...])
        y = _lrelu(_mm(y, w1_r[...]) + b1_r[...])
        y = _lrelu(_mm(y, w2_r[...]) + b2_r[...])
        o0 = _mm(y, w3_r[...]) + b3_r[...]
        o0_ref[...] = o0
        a1_ref[...] = _mm(o0, e1_r[...])
        b1o_ref[...] = _mm(o0, e2_r[...]) + eb_r[...]

    full = lambda a: pl.BlockSpec(a.shape, lambda i: (0,) * a.ndim)
    return pl.pallas_call(
        body,
        grid=(n // t,),
        in_specs=[pl.BlockSpec((t, d0), lambda i: (i, 0)),
                  pl.BlockSpec((t, o), lambda i: (i, 0)),
                  full(fmax), full(fsum), full(w0a), full(w0b), full(w0c),
                  full(w0d), full(b0), full(w1), full(b1), full(w2),
                  full(b2), full(w3), full(b3), full(e1), full(e2),
                  full(eb)],
        out_specs=[pl.BlockSpec((t, o), lambda i: (i, 0)),
                   pl.BlockSpec((t, m1), lambda i: (i, 0)),
                   pl.BlockSpec((t, m1), lambda i: (i, 0))],
        out_shape=[jax.ShapeDtypeStruct((n, o), F32),
                   jax.ShapeDtypeStruct((n, m1), F32),
                   jax.ShapeDtypeStruct((n, m1), F32)],
    )(feat, f1, fmax, fsum, w0a, w0b, w0c, w0d, b0, w1, b1, w2, b2, w3,
      b3, e1, e2, eb)


def _mlp1(o0, f1, fmax, fsum, w):
    n, d0 = o0.shape
    o = f1.shape[1]
    t = _tile(n, 512)
    (w0a, w0b, w0c, w0d, b0, w1, b1, w2, b2, w3, b3) = w

    def body(o0_ref, f1_ref, fmax_ref, fsum_ref, w0a_r, w0b_r, w0c_r,
             w0d_r, b0_r, w1_r, b1_r, w2_r, b2_r, w3_r, b3_r, o1_ref):
        favg = fsum_ref[...] / n
        cst = _mm(fmax_ref[...], w0c_r[...]) + _mm(favg, w0d_r[...])
        y = _lrelu(_mm(o0_ref[...], w0a_r[...]) +
                   _mm(f1_ref[...], w0b_r[...]) + cst + b0_r[...])
        y = _lrelu(_mm(y, w1_r[...]) + b1_r[...])
        y = _lrelu(_mm(y, w2_r[...]) + b2_r[...])
        o1_ref[...] = _mm(y, w3_r[...]) + b3_r[...]

    full = lambda a: pl.BlockSpec(a.shape, lambda i: (0,) * a.ndim)
    return pl.pallas_call(
        body,
        grid=(n // t,),
        in_specs=[pl.BlockSpec((t, d0), lambda i: (i, 0)),
                  pl.BlockSpec((t, o), lambda i: (i, 0)),
                  full(fmax), full(fsum), full(w0a), full(w0b), full(w0c),
                  full(w0d), full(b0), full(w1), full(b1), full(w2),
                  full(b2), full(w3), full(b3)],
        out_specs=pl.BlockSpec((t, o), lambda i: (i, 0)),
        out_shape=jax.ShapeDtypeStruct((n, o), F32),
    )(o0, f1, fmax, fsum, w0a, w0b, w0c, w0d, b0, w1, b1, w2, b2, w3, b3)


def _gather_rows(table, idx):
    nrows, d = table.shape
    m = idx.shape[0]
    info = plsc.get_sparse_core_info()
    nw = info.num_cores * info.num_subcores
    nc = info.num_cores
    bpw = m // nw
    assert bpw * nw == m
    max_rows = max(8, (220 * 1024) // (d * 4))
    chunk = 0
    for cand in range(min(bpw, max_rows), 7, -1):
        if cand % 8 == 0 and bpw % cand == 0:
            chunk = cand
            break
    assert chunk, (bpw, max_rows)
    nck = bpw // chunk

    mesh = plsc.VectorSubcoreMesh(core_axis_name="c", subcore_axis_name="s")

    @functools.partial(
        pl.kernel, mesh=mesh,
        compiler_params=pltpu.CompilerParams(use_tc_tiling_on_sc=False),
        out_type=jax.ShapeDtypeStruct((m, d), F32),
        scratch_types=[pltpu.VMEM((nck, chunk), jnp.int32),
                       pltpu.VMEM((chunk, d), F32),
                       pltpu.SemaphoreType.DMA],
    )
    def gk(table_hbm, idx_hbm, out_hbm, idx_v, rows_v, sem):
        wid = lax.axis_index("s") * nc + lax.axis_index("c")
        pltpu.sync_copy(idx_hbm.at[wid], idx_v)
        base = wid * bpw
        for ck in range(nck):
            pltpu.async_copy(table_hbm.at[idx_v.at[ck]], rows_v, sem).wait()
            pltpu.sync_copy(rows_v,
                            out_hbm.at[pl.ds(base + ck * chunk, chunk)])

    return gk(table, idx.reshape(nw, nck, chunk))


def kernel(feat, idx0, params):
    b, n, d0 = feat.shape
    k = idx0.shape[2]
    eps = 1e-5

    def fold_edge(p, d):
        s = p['bne0_g'] / jnp.sqrt(1.0 + eps)
        w1 = p['ec0_w'][:, :d] * s[:, None]
        w2 = p['ec0_w'][:, d:] * s[:, None]
        return w1.T, (w2 - w1).T, p['bne0_b'][None, :]

    def fold_fc(wname, p, bn):
        if bn is None:
            return p[wname].T, p[wname.replace('_w', '_b')][None, :]
        s = p[bn + '_g'] / jnp.sqrt(1.0 + eps)
        return (p[wname] * s[:, None]).T, p[bn + '_b'][None, :]

    def mlp_weights(p, d, o):
        w0t, b0 = fold_fc('fc0_w', p, 'bn0')
        w1t, b1 = fold_fc('fc1_w', p, 'bn1')
        w2t, b2 = fold_fc('fc2_w', p, 'bn2')
        w3t, b3 = fold_fc('fc3_w', p, None)
        return (w0t[:d], w0t[d:d + o], w0t[d + o:d + 2 * o],
                w0t[d + 2 * o:], b0, w1t, b1, w2t, b2, w3t, b3)

    p0, p1 = params['b0'], params['b1']
    o0_dim, o1_dim = p0['fc3_w'].shape[0], p1['fc3_w'].shape[0]
    x0 = feat[0]

    w1t0, w2t0, eb0 = fold_edge(p0, d0)
    a0, b0n = _node_linear(x0, w1t0, w2t0, eb0)
    idxf0 = idx0[0].T.reshape(-1)
    g0 = _gather_rows(a0, idxf0).reshape(k, n, a0.shape[1])
    f1_0, fm0, fs0 = _edge_combine(g0, b0n, p0['ec1_w'].T,
                                   p0['ec1_b'][None, :])
    w1t1, w2t1, eb1 = fold_edge(p1, o0_dim)
    o0, a1, b1n = _mlp0(x0, f1_0, fm0, fs0, mlp_weights(p0, d0, o0_dim),
                        w1t1, w2t1, eb1)

    idx1 = _knn_topk(o0, k + 1)[:, 1:]
    idxf1 = idx1.T.reshape(-1)
    g1 = _gather_rows(a1, idxf1).reshape(k, n, a1.shape[1])
    f1_1, fm1, fs1 = _edge_combine(g1, b1n, p1['ec1_w'].T,
                                   p1['ec1_b'][None, :])
    o1 = _mlp1(o0, f1_1, fm1, fs1, mlp_weights(p1, o0_dim, o1_dim))

    return jnp.concatenate([o0, o1], axis=1)[None]


# speedup vs baseline: 2.8141x; 1.0472x over previous
import functools

import jax
import jax.numpy as jnp
from jax import lax
from jax.experimental import pallas as pl
from jax.experimental.pallas import tpu as pltpu
from jax.experimental.pallas import tpu_sc as plsc

F32 = jnp.float32


def _mm(a, b):
    return lax.dot_general(a, b, (((1,), (0,)), ((), ())),
                           preferred_element_type=F32)


def _mm_t(a, b):
    return lax.dot_general(a, b, (((1,), (1,)), ((), ())),
                           preferred_element_type=F32)


def _lrelu(x):
    return jnp.where(x >= 0, x, 0.2 * x)


def _tile(n, cap):
    for t in (512, 256, 128, 64, 32, 16, 8):
        if t <= cap and n % t == 0:
            return t
    raise ValueError(f"no tile for {n}")


def _node_linear(x, w1t, w2t, bias):
    n, d = x.shape
    m = w1t.shape[1]
    t = _tile(n, 512)

    def body(x_ref, w1_ref, w2_ref, b_ref, a_ref, bo_ref):
        xt = x_ref[...]
        a_ref[...] = _mm(xt, w1_ref[...])
        bo_ref[...] = _mm(xt, w2_ref[...]) + b_ref[...]

    return pl.pallas_call(
        body,
        grid=(n // t,),
        in_specs=[pl.BlockSpec((t, d), lambda i: (i, 0)),
                  pl.BlockSpec((d, m), lambda i: (0, 0)),
                  pl.BlockSpec((d, m), lambda i: (0, 0)),
                  pl.BlockSpec((1, m), lambda i: (0, 0))],
        out_specs=[pl.BlockSpec((t, m), lambda i: (i, 0)),
                   pl.BlockSpec((t, m), lambda i: (i, 0))],
        out_shape=[jax.ShapeDtypeStruct((n, m), F32),
                   jax.ShapeDtypeStruct((n, m), F32)],
    )(x, w1t, w2t, bias)


def _edge_combine(gath, bnode, ec1t, ec1b):
    kk, n, d2 = gath.shape
    o = ec1t.shape[1]
    t = _tile(n, 512)

    def body(g_ref, b_ref, w_ref, bias_ref, f1_ref, fmax_ref, fsum_ref):
        i = pl.program_id(0)
        bn = b_ref[...]
        w = w_ref[...]
        bias = bias_ref[...]
        acc = None
        for k in range(kk):
            f = _mm(_lrelu(g_ref[k] + bn), w) + bias
            acc = f if acc is None else jnp.maximum(acc, f)
        f1_ref[...] = acc
        tmax = jnp.max(acc, axis=0, keepdims=True)
        tsum = jnp.sum(acc, axis=0, keepdims=True)

        @pl.when(i == 0)
        def _():
            fmax_ref[...] = tmax
            fsum_ref[...] = tsum

        @pl.when(i != 0)
        def _():
            fmax_ref[...] = jnp.maximum(fmax_ref[...], tmax)
            fsum_ref[...] = fsum_ref[...] + tsum

    return pl.pallas_call(
        body,
        grid=(n // t,),
        in_specs=[pl.BlockSpec((kk, t, d2), lambda i: (0, i, 0)),
                  pl.BlockSpec((t, d2), lambda i: (i, 0)),
                  pl.BlockSpec((d2, o), lambda i: (0, 0)),
                  pl.BlockSpec((1, o), lambda i: (0, 0))],
        out_specs=[pl.BlockSpec((t, o), lambda i: (i, 0)),
                   pl.BlockSpec((1, o), lambda i: (0, 0)),
                   pl.BlockSpec((1, o), lambda i: (0, 0))],
        out_shape=[jax.ShapeDtypeStruct((n, o), F32),
                   jax.ShapeDtypeStruct((1, o), F32),
                   jax.ShapeDtypeStruct((1, o), F32)],
    )(gath, bnode, ec1t, ec1b)


def _knn_topk(x, kp1):
    n, d = x.shape
    r = _tile(n, 256)
    neg = -3.0e38
    bigi = 2**30

    def body(xf_ref, xr_ref, out_ref):
        xr = xr_ref[...]
        xxr = jnp.sum(xr * xr, axis=1, keepdims=True)
        xf = xf_ref[...]
        xxc = jnp.sum(xf * xf, axis=1)
        v = 2.0 * _mm_t(xr, xf) - xxr - xxc[None, :]
        iota = lax.broadcasted_iota(jnp.int32, (r, n), 1)
        ni = []
        for _ in range(kp1):
            m = jnp.max(v, axis=1, keepdims=True)
            cand = jnp.where(v == m, iota, bigi)
            j = jnp.min(cand, axis=1, keepdims=True)
            ni.append(j)
            v = jnp.where(iota == j, neg, v)
        out_ref[...] = jnp.concatenate(ni, axis=1)

    return pl.pallas_call(
        body,
        grid=(n // r,),
        in_specs=[pl.BlockSpec((n, d), lambda i: (0, 0)),
                  pl.BlockSpec((r, d), lambda i: (i, 0))],
        out_specs=pl.BlockSpec((r, kp1), lambda i: (i, 0)),
        out_shape=jax.ShapeDtypeStruct((n, kp1), jnp.int32),
    )(x, x)


def _mlp0(feat, f1, fmax, fsum, w, e1, e2, eb):
    n, d0 = feat.shape
    o = f1.shape[1]
    m1 = e1.shape[1]
    t = _tile(n, 512)
    (w0a, w0b, w0c, w0d, b0, w1, b1, w2, b2, w3, b3) = w

    def body(feat_ref, f1_ref, fmax_ref, fsum_ref, w0a_r, w0b_r, w0c_r,
             w0d_r, b0_r, w1_r, b1_r, w2_r, b2_r, w3_r, b3_r, e1_r, e2_r,
             eb_r, o0_ref, a1_ref, b1o_ref):
        favg = fsum_ref[...] / n
        cst = _mm(fmax_ref[...], w0c_r[...]) + _mm(favg, w0d_r[...])
        y = _lrelu(_mm(feat_ref[...], w0a_r[...]) +
                   _mm(f1_ref[...], w0b_r[...]) + cst + b0_r[...])
        y = _lrelu(_mm(y, w1_r[...]) + b1_r[...])
        y = _lrelu(_mm(y, w2_r[...]) + b2_r[...])
        o0 = _mm(y, w3_r[...]) + b3_r[...]
        o0_ref[...] = o0
        a1_ref[...] = _mm(o0, e1_r[...])
        b1o_ref[...] = _mm(o0, e2_r[...]) + eb_r[...]

    full = lambda a: pl.BlockSpec(a.shape, lambda i: (0,) * a.ndim)
    return pl.pallas_call(
        body,
        grid=(n // t,),
        in_specs=[pl.BlockSpec((t, d0), lambda i: (i, 0)),
                  pl.BlockSpec((t, o), lambda i: (i, 0)),
                  full(fmax), full(fsum), full(w0a), full(w0b), full(w0c),
                  full(w0d), full(b0), full(w1), full(b1), full(w2),
                  full(b2), full(w3), full(b3), full(e1), full(e2),
                  full(eb)],
        out_specs=[pl.BlockSpec((t, o), lambda i: (i, 0)),
                   pl.BlockSpec((t, m1), lambda i: (i, 0)),
                   pl.BlockSpec((t, m1), lambda i: (i, 0))],
        out_shape=[jax.ShapeDtypeStruct((n, o), F32),
                   jax.ShapeDtypeStruct((n, m1), F32),
                   jax.ShapeDtypeStruct((n, m1), F32)],
    )(feat, f1, fmax, fsum, w0a, w0b, w0c, w0d, b0, w1, b1, w2, b2, w3,
      b3, e1, e2, eb)


def _mlp1(o0, f1, fmax, fsum, w):
    n, d0 = o0.shape
    o = f1.shape[1]
    t = _tile(n, 512)
    (w0a, w0b, w0c, w0d, b0, w1, b1, w2, b2, w3, b3) = w

    def body(o0_ref, f1_ref, fmax_ref, fsum_ref, w0a_r, w0b_r, w0c_r,
             w0d_r, b0_r, w1_r, b1_r, w2_r, b2_r, w3_r, b3_r, o1_ref):
        favg = fsum_ref[...] / n
        cst = _mm(fmax_ref[...], w0c_r[...]) + _mm(favg, w0d_r[...])
        y = _lrelu(_mm(o0_ref[...], w0a_r[...]) +
                   _mm(f1_ref[...], w0b_r[...]) + cst + b0_r[...])
        y = _lrelu(_mm(y, w1_r[...]) + b1_r[...])
        y = _lrelu(_mm(y, w2_r[...]) + b2_r[...])
        o1_ref[...] = _mm(y, w3_r[...]) + b3_r[...]

    full = lambda a: pl.BlockSpec(a.shape, lambda i: (0,) * a.ndim)
    return pl.pallas_call(
        body,
        grid=(n // t,),
        in_specs=[pl.BlockSpec((t, d0), lambda i: (i, 0)),
                  pl.BlockSpec((t, o), lambda i: (i, 0)),
                  full(fmax), full(fsum), full(w0a), full(w0b), full(w0c),
                  full(w0d), full(b0), full(w1), full(b1), full(w2),
                  full(b2), full(w3), full(b3)],
        out_specs=pl.BlockSpec((t, o), lambda i: (i, 0)),
        out_shape=jax.ShapeDtypeStruct((n, o), F32),
    )(o0, f1, fmax, fsum, w0a, w0b, w0c, w0d, b0, w1, b1, w2, b2, w3, b3)


def _gather_rows(table, idx):
    nrows, d = table.shape
    m = idx.shape[0]
    info = plsc.get_sparse_core_info()
    nw = info.num_cores * info.num_subcores
    nc = info.num_cores
    bpw = m // nw
    assert bpw * nw == m
    max_rows = max(8, (220 * 1024) // (d * 4))
    chunk = 0
    for cand in range(min(bpw, max_rows), 7, -1):
        if cand % 8 == 0 and bpw % cand == 0:
            chunk = cand
            break
    assert chunk, (bpw, max_rows)
    nck = bpw // chunk

    mesh = plsc.VectorSubcoreMesh(core_axis_name="c", subcore_axis_name="s")

    @functools.partial(
        pl.kernel, mesh=mesh,
        compiler_params=pltpu.CompilerParams(use_tc_tiling_on_sc=False),
        out_type=jax.ShapeDtypeStruct((m, d), F32),
        scratch_types=[pltpu.VMEM((nck, chunk), jnp.int32),
                       pltpu.VMEM((chunk, d), F32),
                       pltpu.SemaphoreType.DMA],
    )
    def gk(table_hbm, idx_hbm, out_hbm, idx_v, rows_v, sem):
        wid = lax.axis_index("s") * nc + lax.axis_index("c")
        pltpu.sync_copy(idx_hbm.at[wid], idx_v)
        base = wid * bpw
        for ck in range(nck):
            pltpu.async_copy(table_hbm.at[idx_v.at[ck]], rows_v, sem).wait()
            pltpu.sync_copy(rows_v,
                            out_hbm.at[pl.ds(base + ck * chunk, chunk)])

    return gk(table, idx.reshape(nw, nck, chunk))


def kernel(feat, idx0, params):
    b, n, d0 = feat.shape
    k = idx0.shape[2]
    eps = 1e-5

    def fold_edge(p, d):
        s = p['bne0_g'] / jnp.sqrt(1.0 + eps)
        w1 = p['ec0_w'][:, :d] * s[:, None]
        w2 = p['ec0_w'][:, d:] * s[:, None]
        return w1.T, (w2 - w1).T, p['bne0_b'][None, :]

    def fold_fc(wname, p, bn):
        if bn is None:
            return p[wname].T, p[wname.replace('_w', '_b')][None, :]
        s = p[bn + '_g'] / jnp.sqrt(1.0 + eps)
        return (p[wname] * s[:, None]).T, p[bn + '_b'][None, :]

    def mlp_weights(p, d, o):
        w0t, b0 = fold_fc('fc0_w', p, 'bn0')
        w1t, b1 = fold_fc('fc1_w', p, 'bn1')
        w2t, b2 = fold_fc('fc2_w', p, 'bn2')
        w3t, b3 = fold_fc('fc3_w', p, None)
        return (w0t[:d], w0t[d:d + o], w0t[d + o:d + 2 * o],
                w0t[d + 2 * o:], b0, w1t, b1, w2t, b2, w3t, b3)

    p0, p1 = params['b0'], params['b1']
    o0_dim, o1_dim = p0['fc3_w'].shape[0], p1['fc3_w'].shape[0]
    x0 = feat[0]

    w1t0, w2t0, eb0 = fold_edge(p0, d0)
    a0, b0n = _node_linear(x0, w1t0, w2t0, eb0)
    idxf0 = idx0[0].T.reshape(-1)
    g0 = _gather_rows(a0, idxf0).reshape(k, n, a0.shape[1])
    f1_0, fm0, fs0 = _edge_combine(g0, b0n, p0['ec1_w'].T,
                                   p0['ec1_b'][None, :])
    w1t1, w2t1, eb1 = fold_edge(p1, o0_dim)
    o0, a1, b1n = _mlp0(x0, f1_0, fm0, fs0, mlp_weights(p0, d0, o0_dim),
                        w1t1, w2t1, eb1)

    idx1 = _knn_topk(o0, k + 1)[:, 1:]
    idxf1 = idx1.T.reshape(-1)
    g1 = _gather_rows(a1, idxf1).reshape(k, n, a1.shape[1])
    f1_1, fm1, fs1 = _edge_combine(g1, b1n, p1['ec1_w'].T,
                                   p1['ec1_b'][None, :])
    o1 = _mlp1(o0, f1_1, fm1, fs1, mlp_weights(p1, o0_dim, o1_dim))

    return jnp.concatenate([o0, o1], axis=1)[None]


# speedup vs baseline: 3.3414x; 1.1874x over previous
import functools

import jax
import jax.numpy as jnp
from jax import lax
from jax.experimental import pallas as pl
from jax.experimental.pallas import tpu as pltpu
from jax.experimental.pallas import tpu_sc as plsc

F32 = jnp.float32


def _mm(a, b):
    return lax.dot_general(a, b, (((1,), (0,)), ((), ())),
                           preferred_element_type=F32)


def _mm_t(a, b):
    return lax.dot_general(a, b, (((1,), (1,)), ((), ())),
                           preferred_element_type=F32)


def _lrelu(x):
    return jnp.where(x >= 0, x, 0.2 * x)


def _tile(n, cap):
    for t in (512, 256, 128, 64, 32, 16, 8):
        if t <= cap and n % t == 0:
            return t
    raise ValueError(f"no tile for {n}")


def _node_linear(x, w1t, w2t, bias):
    n, d = x.shape
    m = w1t.shape[1]
    t = _tile(n, 512)

    def body(x_ref, w1_ref, w2_ref, b_ref, a_ref, bo_ref):
        xt = x_ref[...]
        a_ref[...] = _mm(xt, w1_ref[...])
        bo_ref[...] = _mm(xt, w2_ref[...]) + b_ref[...]

    return pl.pallas_call(
        body,
        grid=(n // t,),
        in_specs=[pl.BlockSpec((t, d), lambda i: (i, 0)),
                  pl.BlockSpec((d, m), lambda i: (0, 0)),
                  pl.BlockSpec((d, m), lambda i: (0, 0)),
                  pl.BlockSpec((1, m), lambda i: (0, 0))],
        out_specs=[pl.BlockSpec((t, m), lambda i: (i, 0)),
                   pl.BlockSpec((t, m), lambda i: (i, 0))],
        out_shape=[jax.ShapeDtypeStruct((n, m), F32),
                   jax.ShapeDtypeStruct((n, m), F32)],
    )(x, w1t, w2t, bias)


def _edge_combine(gath, bnode, ec1t, ec1b):
    kk, n, d2 = gath.shape
    o = ec1t.shape[1]
    t = _tile(n, 512)

    def body(g_ref, b_ref, w_ref, bias_ref, f1_ref, fmax_ref, fsum_ref):
        i = pl.program_id(0)
        bn = b_ref[...]
        w = w_ref[...]
        bias = bias_ref[...]
        acc = None
        for k in range(kk):
            f = _mm(_lrelu(g_ref[k] + bn), w) + bias
            acc = f if acc is None else jnp.maximum(acc, f)
        f1_ref[...] = acc
        tmax = jnp.max(acc, axis=0, keepdims=True)
        tsum = jnp.sum(acc, axis=0, keepdims=True)

        @pl.when(i == 0)
        def _():
            fmax_ref[...] = tmax
            fsum_ref[...] = tsum

        @pl.when(i != 0)
        def _():
            fmax_ref[...] = jnp.maximum(fmax_ref[...], tmax)
            fsum_ref[...] = fsum_ref[...] + tsum

    return pl.pallas_call(
        body,
        grid=(n // t,),
        in_specs=[pl.BlockSpec((kk, t, d2), lambda i: (0, i, 0)),
                  pl.BlockSpec((t, d2), lambda i: (i, 0)),
                  pl.BlockSpec((d2, o), lambda i: (0, 0)),
                  pl.BlockSpec((1, o), lambda i: (0, 0))],
        out_specs=[pl.BlockSpec((t, o), lambda i: (i, 0)),
                   pl.BlockSpec((1, o), lambda i: (0, 0)),
                   pl.BlockSpec((1, o), lambda i: (0, 0))],
        out_shape=[jax.ShapeDtypeStruct((n, o), F32),
                   jax.ShapeDtypeStruct((1, o), F32),
                   jax.ShapeDtypeStruct((1, o), F32)],
    )(gath, bnode, ec1t, ec1b)


def _knn_topk(x, xx, kp1):
    n, d = x.shape
    r = _tile(n, 256)
    neg = -3.0e38
    bigf = 3.0e38

    def body(xf_ref, xx_ref, xr_ref, out_ref):
        xr2 = xr_ref[...] * 2.0
        one = jnp.ones((1, 1), F32)
        xxr_row = _mm_t(one, xx_ref[...])
        v = _mm_t(xr2, xf_ref[...]) - xxr_row
        iota = lax.broadcasted_iota(jnp.int32, (r, n), 1).astype(F32)
        ni = []
        for t in range(kp1):
            m = jnp.max(v, axis=1, keepdims=True)
            cand = jnp.where(v == m, iota, bigf)
            j = jnp.min(cand, axis=1, keepdims=True)
            ni.append(j)
            if t + 1 < kp1:
                v = jnp.where(iota == j, neg, v)
        out_ref[...] = jnp.concatenate(ni, axis=1).astype(jnp.int32)

    return pl.pallas_call(
        body,
        grid=(n // r,),
        in_specs=[pl.BlockSpec((n, d), lambda i: (0, 0)),
                  pl.BlockSpec((n, 1), lambda i: (0, 0)),
                  pl.BlockSpec((r, d), lambda i: (i, 0))],
        out_specs=pl.BlockSpec((r, kp1), lambda i: (i, 0)),
        out_shape=jax.ShapeDtypeStruct((n, kp1), jnp.int32),
    )(x, xx, x)


def _mlp0(feat, f1, fmax, fsum, w, e1, e2, eb):
    n, d0 = feat.shape
    o = f1.shape[1]
    m1 = e1.shape[1]
    t = _tile(n, 512)
    (w0a, w0b, w0c, w0d, b0, w1, b1, w2, b2, w3, b3) = w

    def body(feat_ref, f1_ref, fmax_ref, fsum_ref, w0a_r, w0b_r, w0c_r,
             w0d_r, b0_r, w1_r, b1_r, w2_r, b2_r, w3_r, b3_r, e1_r, e2_r,
             eb_r, o0_ref, a1_ref, b1o_ref, xx_ref):
        favg = fsum_ref[...] / n
        cst = _mm(fmax_ref[...], w0c_r[...]) + _mm(favg, w0d_r[...])
        y = _lrelu(_mm(feat_ref[...], w0a_r[...]) +
                   _mm(f1_ref[...], w0b_r[...]) + cst + b0_r[...])
        y = _lrelu(_mm(y, w1_r[...]) + b1_r[...])
        y = _lrelu(_mm(y, w2_r[...]) + b2_r[...])
        o0 = _mm(y, w3_r[...]) + b3_r[...]
        o0_ref[...] = o0
        a1_ref[...] = _mm(o0, e1_r[...])
        b1o_ref[...] = _mm(o0, e2_r[...]) + eb_r[...]
        xx_ref[...] = jnp.sum(o0 * o0, axis=1, keepdims=True)

    full = lambda a: pl.BlockSpec(a.shape, lambda i: (0,) * a.ndim)
    return pl.pallas_call(
        body,
        grid=(n // t,),
        in_specs=[pl.BlockSpec((t, d0), lambda i: (i, 0)),
                  pl.BlockSpec((t, o), lambda i: (i, 0)),
                  full(fmax), full(fsum), full(w0a), full(w0b), full(w0c),
                  full(w0d), full(b0), full(w1), full(b1), full(w2),
                  full(b2), full(w3), full(b3), full(e1), full(e2),
                  full(eb)],
        out_specs=[pl.BlockSpec((t, o), lambda i: (i, 0)),
                   pl.BlockSpec((t, m1), lambda i: (i, 0)),
                   pl.BlockSpec((t, m1), lambda i: (i, 0)),
                   pl.BlockSpec((t, 1), lambda i: (i, 0))],
        out_shape=[jax.ShapeDtypeStruct((n, o), F32),
                   jax.ShapeDtypeStruct((n, m1), F32),
                   jax.ShapeDtypeStruct((n, m1), F32),
                   jax.ShapeDtypeStruct((n, 1), F32)],
    )(feat, f1, fmax, fsum, w0a, w0b, w0c, w0d, b0, w1, b1, w2, b2, w3,
      b3, e1, e2, eb)


def _mlp1(o0, f1, fmax, fsum, w):
    n, d0 = o0.shape
    o = f1.shape[1]
    t = _tile(n, 512)
    (w0a, w0b, w0c, w0d, b0, w1, b1, w2, b2, w3, b3) = w

    def body(o0_ref, f1_ref, fmax_ref, fsum_ref, w0a_r, w0b_r, w0c_r,
             w0d_r, b0_r, w1_r, b1_r, w2_r, b2_r, w3_r, b3_r, o1_ref):
        favg = fsum_ref[...] / n
        cst = _mm(fmax_ref[...], w0c_r[...]) + _mm(favg, w0d_r[...])
        y = _lrelu(_mm(o0_ref[...], w0a_r[...]) +
                   _mm(f1_ref[...], w0b_r[...]) + cst + b0_r[...])
        y = _lrelu(_mm(y, w1_r[...]) + b1_r[...])
        y = _lrelu(_mm(y, w2_r[...]) + b2_r[...])
        o1_ref[...] = _mm(y, w3_r[...]) + b3_r[...]

    full = lambda a: pl.BlockSpec(a.shape, lambda i: (0,) * a.ndim)
    return pl.pallas_call(
        body,
        grid=(n // t,),
        in_specs=[pl.BlockSpec((t, d0), lambda i: (i, 0)),
                  pl.BlockSpec((t, o), lambda i: (i, 0)),
                  full(fmax), full(fsum), full(w0a), full(w0b), full(w0c),
                  full(w0d), full(b0), full(w1), full(b1), full(w2),
                  full(b2), full(w3), full(b3)],
        out_specs=pl.BlockSpec((t, o), lambda i: (i, 0)),
        out_shape=jax.ShapeDtypeStruct((n, o), F32),
    )(o0, f1, fmax, fsum, w0a, w0b, w0c, w0d, b0, w1, b1, w2, b2, w3, b3)


def _gather_rows(table, idx):
    nrows, d = table.shape
    m = idx.shape[0]
    info = plsc.get_sparse_core_info()
    nw = info.num_cores * info.num_subcores
    nc = info.num_cores
    bpw = m // nw
    assert bpw * nw == m
    max_rows = max(8, (220 * 1024) // (d * 4))
    chunk = 0
    for cand in range(min(bpw, max_rows), 7, -1):
        if cand % 8 == 0 and bpw % cand == 0:
            chunk = cand
            break
    assert chunk, (bpw, max_rows)
    nck = bpw // chunk

    mesh = plsc.VectorSubcoreMesh(core_axis_name="c", subcore_axis_name="s")

    @functools.partial(
        pl.kernel, mesh=mesh,
        compiler_params=pltpu.CompilerParams(use_tc_tiling_on_sc=False),
        out_type=jax.ShapeDtypeStruct((m, d), F32),
        scratch_types=[pltpu.VMEM((nck, chunk), jnp.int32),
                       pltpu.VMEM((chunk, d), F32),
                       pltpu.SemaphoreType.DMA],
    )
    def gk(table_hbm, idx_hbm, out_hbm, idx_v, rows_v, sem):
        wid = lax.axis_index("s") * nc + lax.axis_index("c")
        pltpu.sync_copy(idx_hbm.at[wid], idx_v)
        base = wid * bpw
        for ck in range(nck):
            pltpu.async_copy(table_hbm.at[idx_v.at[ck]], rows_v, sem).wait()
            pltpu.sync_copy(rows_v,
                            out_hbm.at[pl.ds(base + ck * chunk, chunk)])

    return gk(table, idx.reshape(nw, nck, chunk))


def kernel(feat, idx0, params):
    b, n, d0 = feat.shape
    k = idx0.shape[2]
    eps = 1e-5

    def fold_edge(p, d):
        s = p['bne0_g'] / jnp.sqrt(1.0 + eps)
        w1 = p['ec0_w'][:, :d] * s[:, None]
        w2 = p['ec0_w'][:, d:] * s[:, None]
        return w1.T, (w2 - w1).T, p['bne0_b'][None, :]

    def fold_fc(wname, p, bn):
        if bn is None:
            return p[wname].T, p[wname.replace('_w', '_b')][None, :]
        s = p[bn + '_g'] / jnp.sqrt(1.0 + eps)
        return (p[wname] * s[:, None]).T, p[bn + '_b'][None, :]

    def mlp_weights(p, d, o):
        w0t, b0 = fold_fc('fc0_w', p, 'bn0')
        w1t, b1 = fold_fc('fc1_w', p, 'bn1')
        w2t, b2 = fold_fc('fc2_w', p, 'bn2')
        w3t, b3 = fold_fc('fc3_w', p, None)
        return (w0t[:d], w0t[d:d + o], w0t[d + o:d + 2 * o],
                w0t[d + 2 * o:], b0, w1t, b1, w2t, b2, w3t, b3)

    p0, p1 = params['b0'], params['b1']
    o0_dim, o1_dim = p0['fc3_w'].shape[0], p1['fc3_w'].shape[0]
    x0 = feat[0]

    w1t0, w2t0, eb0 = fold_edge(p0, d0)
    a0, b0n = _node_linear(x0, w1t0, w2t0, eb0)
    idxf0 = idx0[0].T.reshape(-1)
    g0 = _gather_rows(a0, idxf0).reshape(k, n, a0.shape[1])
    f1_0, fm0, fs0 = _edge_combine(g0, b0n, p0['ec1_w'].T,
                                   p0['ec1_b'][None, :])
    w1t1, w2t1, eb1 = fold_edge(p1, o0_dim)
    o0, a1, b1n, xx0 = _mlp0(x0, f1_0, fm0, fs0,
                             mlp_weights(p0, d0, o0_dim), w1t1, w2t1, eb1)

    idx1 = _knn_topk(o0, xx0, k + 1)[:, 1:]
    idxf1 = idx1.T.reshape(-1)
    g1 = _gather_rows(a1, idxf1).reshape(k, n, a1.shape[1])
    f1_1, fm1, fs1 = _edge_combine(g1, b1n, p1['ec1_w'].T,
                                   p1['ec1_b'][None, :])
    o1 = _mlp1(o0, f1_1, fm1, fs1, mlp_weights(p1, o0_dim, o1_dim))

    return jnp.concatenate([o0, o1], axis=1)[None]


# speedup vs baseline: 4.0685x; 1.2176x over previous
import functools

import jax
import jax.numpy as jnp
from jax import lax
from jax.experimental import pallas as pl
from jax.experimental.pallas import tpu as pltpu
from jax.experimental.pallas import tpu_sc as plsc

F32 = jnp.float32


def _mm(a, b):
    return lax.dot_general(a, b, (((1,), (0,)), ((), ())),
                           preferred_element_type=F32)


def _mm_t(a, b):
    return lax.dot_general(a, b, (((1,), (1,)), ((), ())),
                           preferred_element_type=F32)


def _lrelu(x):
    return jnp.where(x >= 0, x, 0.2 * x)


def _tile(n, cap):
    for t in (512, 256, 128, 64, 32, 16, 8):
        if t <= cap and n % t == 0:
            return t
    raise ValueError(f"no tile for {n}")


def _node_linear(x, w1t, w2t, bias):
    n, d = x.shape
    m = w1t.shape[1]
    t = _tile(n, 512)

    def body(x_ref, w1_ref, w2_ref, b_ref, a_ref, bo_ref):
        xt = x_ref[...]
        a_ref[...] = _mm(xt, w1_ref[...])
        bo_ref[...] = _mm(xt, w2_ref[...]) + b_ref[...]

    return pl.pallas_call(
        body,
        grid=(n // t,),
        in_specs=[pl.BlockSpec((t, d), lambda i: (i, 0)),
                  pl.BlockSpec((d, m), lambda i: (0, 0)),
                  pl.BlockSpec((d, m), lambda i: (0, 0)),
                  pl.BlockSpec((1, m), lambda i: (0, 0))],
        out_specs=[pl.BlockSpec((t, m), lambda i: (i, 0)),
                   pl.BlockSpec((t, m), lambda i: (i, 0))],
        out_shape=[jax.ShapeDtypeStruct((n, m), F32),
                   jax.ShapeDtypeStruct((n, m), F32)],
    )(x, w1t, w2t, bias)


def _edge_combine(gath, bnode, ec1t, ec1b):
    kk, n, d2 = gath.shape
    o = ec1t.shape[1]
    t = _tile(n, 512)

    def body(g_ref, b_ref, w_ref, bias_ref, f1_ref, fmax_ref, fsum_ref):
        i = pl.program_id(0)
        bn = b_ref[...]
        w = w_ref[...]
        bias = bias_ref[...]
        acc = None
        for k in range(kk):
            f = _mm(_lrelu(g_ref[k] + bn), w) + bias
            acc = f if acc is None else jnp.maximum(acc, f)
        f1_ref[...] = acc
        tmax = jnp.max(acc, axis=0, keepdims=True)
        tsum = jnp.sum(acc, axis=0, keepdims=True)

        @pl.when(i == 0)
        def _():
            fmax_ref[...] = tmax
            fsum_ref[...] = tsum

        @pl.when(i != 0)
        def _():
            fmax_ref[...] = jnp.maximum(fmax_ref[...], tmax)
            fsum_ref[...] = fsum_ref[...] + tsum

    return pl.pallas_call(
        body,
        grid=(n // t,),
        in_specs=[pl.BlockSpec((kk, t, d2), lambda i: (0, i, 0)),
                  pl.BlockSpec((t, d2), lambda i: (i, 0)),
                  pl.BlockSpec((d2, o), lambda i: (0, 0)),
                  pl.BlockSpec((1, o), lambda i: (0, 0))],
        out_specs=[pl.BlockSpec((t, o), lambda i: (i, 0)),
                   pl.BlockSpec((1, o), lambda i: (0, 0)),
                   pl.BlockSpec((1, o), lambda i: (0, 0))],
        out_shape=[jax.ShapeDtypeStruct((n, o), F32),
                   jax.ShapeDtypeStruct((1, o), F32),
                   jax.ShapeDtypeStruct((1, o), F32)],
    )(gath, bnode, ec1t, ec1b)


def _knn_topk(x, xx, kp1):
    n, d = x.shape
    r = _tile(n, 256)
    neg = -3.0e38
    bigf = 3.0e38

    nt = n // 128
    nlev = kp1 - 2 if n // 128 < kp1 else 4

    def body(xf_ref, xx_ref, xr_ref, out_ref):
        xr2 = xr_ref[...] * 2.0
        one = jnp.ones((1, 1), F32)
        xxr_row = _mm_t(one, xx_ref[...])
        v = _mm_t(xr2, xf_ref[...]) - xxr_row
        vs = [v[:, t * 128:(t + 1) * 128] for t in range(nt)]
        ms, js = [], []
        for lev in range(nlev):
            mk = vs[0]
            for t in range(1, nt):
                mk = jnp.maximum(mk, vs[t])
            jk = jnp.full((r, 128), bigf, F32)
            for t in range(nt):
                jk = jnp.minimum(jk, jnp.where(vs[t] == mk, float(t), bigf))
            ms.append(mk)
            js.append(jk)
            if lev + 1 < nlev:
                for t in range(nt):
                    vs[t] = jnp.where(jk == float(t), neg, vs[t])
        lane = lax.broadcasted_iota(jnp.int32, (r, 128), 1).astype(F32)
        vv = jnp.concatenate(ms, axis=1)
        ii = jnp.concatenate([jk * 128.0 + lane for jk in js], axis=1)
        pos = lax.broadcasted_iota(jnp.int32, (r, nlev * 128), 1).astype(F32)
        ni = []
        flag = jnp.zeros((r, 1), jnp.bool_)
        for t in range(kp1):
            m = jnp.max(vv, axis=1, keepdims=True)
            cand = jnp.where(vv == m, ii, bigf)
            j = jnp.min(cand, axis=1, keepdims=True)
            ni.append(j)
            if t + 1 < kp1:
                p = jnp.min(jnp.where(ii == j, pos, bigf), axis=1,
                            keepdims=True)
                flag = flag | (p >= (nlev - 1) * 128.0)
                vv = jnp.where(ii == j, neg, vv)
        out_ref[...] = jnp.concatenate(ni, axis=1).astype(jnp.int32)

        @pl.when(jnp.any(flag))
        def _():
            v2 = _mm_t(xr2, xf_ref[...]) - xxr_row
            iota = lax.broadcasted_iota(jnp.int32, (r, n), 1).astype(F32)
            ni2 = []
            for t in range(kp1):
                m = jnp.max(v2, axis=1, keepdims=True)
                cand = jnp.where(v2 == m, iota, bigf)
                j = jnp.min(cand, axis=1, keepdims=True)
                ni2.append(j)
                if t + 1 < kp1:
                    v2 = jnp.where(iota == j, neg, v2)
            out_ref[...] = jnp.concatenate(ni2, axis=1).astype(jnp.int32)

    return pl.pallas_call(
        body,
        grid=(n // r,),
        in_specs=[pl.BlockSpec((n, d), lambda i: (0, 0)),
                  pl.BlockSpec((n, 1), lambda i: (0, 0)),
                  pl.BlockSpec((r, d), lambda i: (i, 0))],
        out_specs=pl.BlockSpec((r, kp1), lambda i: (i, 0)),
        out_shape=jax.ShapeDtypeStruct((n, kp1), jnp.int32),
    )(x, xx, x)


def _mlp0(feat, f1, fmax, fsum, w, e1, e2, eb):
    n, d0 = feat.shape
    o = f1.shape[1]
    m1 = e1.shape[1]
    t = _tile(n, 512)
    (w0a, w0b, w0c, w0d, b0, w1, b1, w2, b2, w3, b3) = w

    def body(feat_ref, f1_ref, fmax_ref, fsum_ref, w0a_r, w0b_r, w0c_r,
             w0d_r, b0_r, w1_r, b1_r, w2_r, b2_r, w3_r, b3_r, e1_r, e2_r,
             eb_r, o0_ref, a1_ref, b1o_ref, xx_ref):
        favg = fsum_ref[...] / n
        cst = _mm(fmax_ref[...], w0c_r[...]) + _mm(favg, w0d_r[...])
        y = _lrelu(_mm(feat_ref[...], w0a_r[...]) +
                   _mm(f1_ref[...], w0b_r[...]) + cst + b0_r[...])
        y = _lrelu(_mm(y, w1_r[...]) + b1_r[...])
        y = _lrelu(_mm(y, w2_r[...]) + b2_r[...])
        o0 = _mm(y, w3_r[...]) + b3_r[...]
        o0_ref[...] = o0
        a1_ref[...] = _mm(o0, e1_r[...])
        b1o_ref[...] = _mm(o0, e2_r[...]) + eb_r[...]
        xx_ref[...] = jnp.sum(o0 * o0, axis=1, keepdims=True)

    full = lambda a: pl.BlockSpec(a.shape, lambda i: (0,) * a.ndim)
    return pl.pallas_call(
        body,
        grid=(n // t,),
        in_specs=[pl.BlockSpec((t, d0), lambda i: (i, 0)),
                  pl.BlockSpec((t, o), lambda i: (i, 0)),
                  full(fmax), full(fsum), full(w0a), full(w0b), full(w0c),
                  full(w0d), full(b0), full(w1), full(b1), full(w2),
                  full(b2), full(w3), full(b3), full(e1), full(e2),
                  full(eb)],
        out_specs=[pl.BlockSpec((t, o), lambda i: (i, 0)),
                   pl.BlockSpec((t, m1), lambda i: (i, 0)),
                   pl.BlockSpec((t, m1), lambda i: (i, 0)),
                   pl.BlockSpec((t, 1), lambda i: (i, 0))],
        out_shape=[jax.ShapeDtypeStruct((n, o), F32),
                   jax.ShapeDtypeStruct((n, m1), F32),
                   jax.ShapeDtypeStruct((n, m1), F32),
                   jax.ShapeDtypeStruct((n, 1), F32)],
    )(feat, f1, fmax, fsum, w0a, w0b, w0c, w0d, b0, w1, b1, w2, b2, w3,
      b3, e1, e2, eb)


def _mlp1(o0, f1, fmax, fsum, w):
    n, d0 = o0.shape
    o = f1.shape[1]
    t = _tile(n, 512)
    (w0a, w0b, w0c, w0d, b0, w1, b1, w2, b2, w3, b3) = w

    def body(o0_ref, f1_ref, fmax_ref, fsum_ref, w0a_r, w0b_r, w0c_r,
             w0d_r, b0_r, w1_r, b1_r, w2_r, b2_r, w3_r, b3_r, o1_ref):
        favg = fsum_ref[...] / n
        cst = _mm(fmax_ref[...], w0c_r[...]) + _mm(favg, w0d_r[...])
        y = _lrelu(_mm(o0_ref[...], w0a_r[...]) +
                   _mm(f1_ref[...], w0b_r[...]) + cst + b0_r[...])
        y = _lrelu(_mm(y, w1_r[...]) + b1_r[...])
        y = _lrelu(_mm(y, w2_r[...]) + b2_r[...])
        o1_ref[...] = _mm(y, w3_r[...]) + b3_r[...]

    full = lambda a: pl.BlockSpec(a.shape, lambda i: (0,) * a.ndim)
    return pl.pallas_call(
        body,
        grid=(n // t,),
        in_specs=[pl.BlockSpec((t, d0), lambda i: (i, 0)),
                  pl.BlockSpec((t, o), lambda i: (i, 0)),
                  full(fmax), full(fsum), full(w0a), full(w0b), full(w0c),
                  full(w0d), full(b0), full(w1), full(b1), full(w2),
                  full(b2), full(w3), full(b3)],
        out_specs=pl.BlockSpec((t, o), lambda i: (i, 0)),
        out_shape=jax.ShapeDtypeStruct((n, o), F32),
    )(o0, f1, fmax, fsum, w0a, w0b, w0c, w0d, b0, w1, b1, w2, b2, w3, b3)


def _gather_rows(table, idx):
    nrows, d = table.shape
    m = idx.shape[0]
    info = plsc.get_sparse_core_info()
    nw = info.num_cores * info.num_subcores
    nc = info.num_cores
    bpw = m // nw
    assert bpw * nw == m
    max_rows = max(8, (220 * 1024) // (d * 4))
    chunk = 0
    for cand in range(min(bpw, max_rows), 7, -1):
        if cand % 8 == 0 and bpw % cand == 0:
            chunk = cand
            break
    assert chunk, (bpw, max_rows)
    nck = bpw // chunk

    mesh = plsc.VectorSubcoreMesh(core_axis_name="c", subcore_axis_name="s")

    @functools.partial(
        pl.kernel, mesh=mesh,
        compiler_params=pltpu.CompilerParams(use_tc_tiling_on_sc=False),
        out_type=jax.ShapeDtypeStruct((m, d), F32),
        scratch_types=[pltpu.VMEM((nck, chunk), jnp.int32),
                       pltpu.VMEM((chunk, d), F32),
                       pltpu.SemaphoreType.DMA],
    )
    def gk(table_hbm, idx_hbm, out_hbm, idx_v, rows_v, sem):
        wid = lax.axis_index("s") * nc + lax.axis_index("c")
        pltpu.sync_copy(idx_hbm.at[wid], idx_v)
        base = wid * bpw
        for ck in range(nck):
            pltpu.async_copy(table_hbm.at[idx_v.at[ck]], rows_v, sem).wait()
            pltpu.sync_copy(rows_v,
                            out_hbm.at[pl.ds(base + ck * chunk, chunk)])

    return gk(table, idx.reshape(nw, nck, chunk))


def kernel(feat, idx0, params):
    b, n, d0 = feat.shape
    k = idx0.shape[2]
    eps = 1e-5

    def fold_edge(p, d):
        s = p['bne0_g'] / jnp.sqrt(1.0 + eps)
        w1 = p['ec0_w'][:, :d] * s[:, None]
        w2 = p['ec0_w'][:, d:] * s[:, None]
        return w1.T, (w2 - w1).T, p['bne0_b'][None, :]

    def fold_fc(wname, p, bn):
        if bn is None:
            return p[wname].T, p[wname.replace('_w', '_b')][None, :]
        s = p[bn + '_g'] / jnp.sqrt(1.0 + eps)
        return (p[wname] * s[:, None]).T, p[bn + '_b'][None, :]

    def mlp_weights(p, d, o):
        w0t, b0 = fold_fc('fc0_w', p, 'bn0')
        w1t, b1 = fold_fc('fc1_w', p, 'bn1')
        w2t, b2 = fold_fc('fc2_w', p, 'bn2')
        w3t, b3 = fold_fc('fc3_w', p, None)
        return (w0t[:d], w0t[d:d + o], w0t[d + o:d + 2 * o],
                w0t[d + 2 * o:], b0, w1t, b1, w2t, b2, w3t, b3)

    p0, p1 = params['b0'], params['b1']
    o0_dim, o1_dim = p0['fc3_w'].shape[0], p1['fc3_w'].shape[0]
    x0 = feat[0]

    w1t0, w2t0, eb0 = fold_edge(p0, d0)
    a0, b0n = _node_linear(x0, w1t0, w2t0, eb0)
    idxf0 = idx0[0].T.reshape(-1)
    g0 = _gather_rows(a0, idxf0).reshape(k, n, a0.shape[1])
    f1_0, fm0, fs0 = _edge_combine(g0, b0n, p0['ec1_w'].T,
                                   p0['ec1_b'][None, :])
    w1t1, w2t1, eb1 = fold_edge(p1, o0_dim)
    o0, a1, b1n, xx0 = _mlp0(x0, f1_0, fm0, fs0,
                             mlp_weights(p0, d0, o0_dim), w1t1, w2t1, eb1)

    idx1 = _knn_topk(o0, xx0, k + 1)[:, 1:]
    idxf1 = idx1.T.reshape(-1)
    g1 = _gather_rows(a1, idxf1).reshape(k, n, a1.shape[1])
    f1_1, fm1, fs1 = _edge_combine(g1, b1n, p1['ec1_w'].T,
                                   p1['ec1_b'][None, :])
    o1 = _mlp1(o0, f1_1, fm1, fs1, mlp_weights(p1, o0_dim, o1_dim))

    return jnp.concatenate([o0, o1], axis=1)[None]
